# baseline (speedup 1.0000x reference)
import functools

import jax
import jax.numpy as jnp
from jax.experimental import pallas as pl
from jax.experimental.pallas import tpu as pltpu

INPUT_DIM = 2048
NUM_EXPERTS = 64
TOP_K = 8
BATCH = 16384
BLOCK_ROWS = 2048

_NEG_INF = float("-inf")


def _router_kernel(x_ref, w_ref, b_ref, wts_ref, idx_ref, lb_ref, z_ref,
                   f_acc, p_acc, z_acc):
    i = pl.program_id(0)
    n = pl.num_programs(0)

    logits = jax.lax.dot_general(
        w_ref[...], x_ref[...],
        dimension_numbers=(((1,), (1,)), ((), ())),
        preferred_element_type=jnp.float32,
    ) + b_ref[...]

    br = logits.shape[1]
    expert_f = jax.lax.broadcasted_iota(
        jnp.int32, (NUM_EXPERTS, br), 0).astype(jnp.float32)

    cur = logits
    vals = []
    idxs_f = []
    hit0 = None
    for j in range(TOP_K):
        m = jnp.max(cur, axis=0, keepdims=True)
        hit = cur == m
        ij = jnp.min(jnp.where(hit, expert_f, float(NUM_EXPERTS)),
                     axis=0, keepdims=True)
        first = expert_f == ij
        if j == 0:
            hit0 = first
        vals.append(m)
        idxs_f.append(ij)
        cur = jnp.where(first, _NEG_INF, cur)

    top_vals = jnp.concatenate(vals, axis=0)
    top_idx_f = jnp.concatenate(idxs_f, axis=0)

    row_max = vals[0]
    e_top = jnp.exp(top_vals - row_max)
    wts = e_top / jnp.sum(e_top, axis=0, keepdims=True)
    wts_ref[...] = wts.T
    idx_ref[...] = top_idx_f.T.astype(jnp.int32)

    e_all = jnp.exp(logits - row_max)
    denom = jnp.sum(e_all, axis=0, keepdims=True)
    p_part = e_all * (1.0 / denom)
    lse = row_max + jnp.log(denom)

    @pl.when(i == 0)
    def _init():
        f_acc[...] = hit0.astype(jnp.float32)
        p_acc[...] = p_part
        z_acc[...] = lse * lse

    @pl.when(i != 0)
    def _accum():
        f_acc[...] += hit0.astype(jnp.float32)
        p_acc[...] += p_part
        z_acc[...] += lse * lse

    @pl.when(i == n - 1)
    def _finalize():
        inv_b = 1.0 / BATCH
        fe = jnp.sum(f_acc[...], axis=1, keepdims=True)
        pe = jnp.sum(p_acc[...], axis=1, keepdims=True)
        lb_ref[...] = (float(NUM_EXPERTS) * inv_b * inv_b
                       * jnp.sum(fe * pe, keepdims=True))
        z_ref[...] = jnp.sum(z_acc[...], axis=1,
                             keepdims=True) * inv_b


@jax.jit
def kernel(x, w_gate, b_gate):
    b2 = b_gate.reshape(NUM_EXPERTS, 1)
    grid = (BATCH // BLOCK_ROWS,)
    wts, idx, lb, z = pl.pallas_call(
        _router_kernel,
        grid=grid,
        in_specs=[
            pl.BlockSpec((BLOCK_ROWS, INPUT_DIM), lambda i: (i, 0)),
            pl.BlockSpec((NUM_EXPERTS, INPUT_DIM), lambda i: (0, 0)),
            pl.BlockSpec((NUM_EXPERTS, 1), lambda i: (0, 0)),
        ],
        out_specs=[
            pl.BlockSpec((BLOCK_ROWS, TOP_K), lambda i: (i, 0)),
            pl.BlockSpec((BLOCK_ROWS, TOP_K), lambda i: (i, 0)),
            pl.BlockSpec((1, 1), lambda i: (0, 0)),
            pl.BlockSpec((1, 1), lambda i: (0, 0)),
        ],
        out_shape=[
            jax.ShapeDtypeStruct((BATCH, TOP_K), jnp.float32),
            jax.ShapeDtypeStruct((BATCH, TOP_K), jnp.int32),
            jax.ShapeDtypeStruct((1, 1), jnp.float32),
            jax.ShapeDtypeStruct((1, 1), jnp.float32),
        ],
        scratch_shapes=[
            pltpu.VMEM((NUM_EXPERTS, BLOCK_ROWS), jnp.float32),
            pltpu.VMEM((NUM_EXPERTS, BLOCK_ROWS), jnp.float32),
            pltpu.VMEM((1, BLOCK_ROWS), jnp.float32),
        ],
    )(x, w_gate, b2)
    return wts, idx, lb[0, 0], z[0, 0]


# speedup vs baseline: 1.3496x; 1.3496x over previous
import functools

import jax
import jax.numpy as jnp
from jax.experimental import pallas as pl
from jax.experimental.pallas import tpu as pltpu

INPUT_DIM = 2048
NUM_EXPERTS = 64
TOP_K = 8
BATCH = 16384
BLOCK_ROWS = 2048

_NEG_INF = float("-inf")


def _router_kernel(x_ref, w_ref, b_ref, wts_ref, idx_ref, lb_ref, z_ref,
                   f_acc, p_acc, z_acc):
    i = pl.program_id(0)
    n = pl.num_programs(0)

    logits = jax.lax.dot_general(
        w_ref[...], x_ref[...],
        dimension_numbers=(((1,), (1,)), ((), ())),
        preferred_element_type=jnp.float32,
    ) + b_ref[...]

    br = logits.shape[1]
    expert_f = jax.lax.broadcasted_iota(
        jnp.int32, (NUM_EXPERTS, br), 0).astype(jnp.float32)

    cur = logits
    vals = []
    idxs_f = []
    hit0 = None
    for j in range(TOP_K):
        m = jnp.max(cur, axis=0, keepdims=True)
        hit = cur == m
        ij = jnp.min(jnp.where(hit, expert_f, float(NUM_EXPERTS)),
                     axis=0, keepdims=True)
        first = expert_f == ij
        if j == 0:
            hit0 = first
        vals.append(m)
        idxs_f.append(ij)
        cur = jnp.where(first, _NEG_INF, cur)

    top_vals = jnp.concatenate(vals, axis=0)
    top_idx = jnp.concatenate(idxs_f, axis=0).astype(jnp.int32)

    row_max = vals[0]
    e_top = jnp.exp(top_vals - row_max)
    wts_ref[...] = e_top / jnp.sum(e_top, axis=0, keepdims=True)
    idx_ref[...] = top_idx

    e_all = jnp.exp(logits - row_max)
    denom = jnp.sum(e_all, axis=0, keepdims=True)
    p_part = e_all * (1.0 / denom)
    lse = row_max + jnp.log(denom)

    @pl.when(i == 0)
    def _init():
        f_acc[...] = hit0.astype(jnp.float32)
        p_acc[...] = p_part
        z_acc[...] = lse * lse

    @pl.when(i != 0)
    def _accum():
        f_acc[...] += hit0.astype(jnp.float32)
        p_acc[...] += p_part
        z_acc[...] += lse * lse

    @pl.when(i == n - 1)
    def _finalize():
        inv_b = 1.0 / BATCH
        fe = jnp.sum(f_acc[...], axis=1, keepdims=True)
        pe = jnp.sum(p_acc[...], axis=1, keepdims=True)
        lb_ref[...] = (float(NUM_EXPERTS) * inv_b * inv_b
                       * jnp.sum(fe * pe, keepdims=True))
        z_ref[...] = jnp.sum(z_acc[...], axis=1,
                             keepdims=True) * inv_b


@jax.jit
def kernel(x, w_gate, b_gate):
    b2 = b_gate.reshape(NUM_EXPERTS, 1)
    grid = (BATCH // BLOCK_ROWS,)
    wts_t, idx_t, lb, z = pl.pallas_call(
        _router_kernel,
        grid=grid,
        in_specs=[
            pl.BlockSpec((BLOCK_ROWS, INPUT_DIM), lambda i: (i, 0)),
            pl.BlockSpec((NUM_EXPERTS, INPUT_DIM), lambda i: (0, 0)),
            pl.BlockSpec((NUM_EXPERTS, 1), lambda i: (0, 0)),
        ],
        out_specs=[
            pl.BlockSpec((TOP_K, BLOCK_ROWS), lambda i: (0, i)),
            pl.BlockSpec((TOP_K, BLOCK_ROWS), lambda i: (0, i)),
            pl.BlockSpec((1, 1), lambda i: (0, 0)),
            pl.BlockSpec((1, 1), lambda i: (0, 0)),
        ],
        out_shape=[
            jax.ShapeDtypeStruct((TOP_K, BATCH), jnp.float32),
            jax.ShapeDtypeStruct((TOP_K, BATCH), jnp.int32),
            jax.ShapeDtypeStruct((1, 1), jnp.float32),
            jax.ShapeDtypeStruct((1, 1), jnp.float32),
        ],
        scratch_shapes=[
            pltpu.VMEM((NUM_EXPERTS, BLOCK_ROWS), jnp.float32),
            pltpu.VMEM((NUM_EXPERTS, BLOCK_ROWS), jnp.float32),
            pltpu.VMEM((1, BLOCK_ROWS), jnp.float32),
        ],
    )(x, w_gate, b2)
    return wts_t.T, idx_t.T, lb[0, 0], z[0, 0]
